# baseline (speedup 1.0000x reference)
import functools

import jax
import jax.numpy as jnp
from jax import lax
from jax.experimental import pallas as pl
from jax.experimental.pallas import tpu as pltpu
from jax.experimental.pallas import tpu_sc as plsc

N = 10000
E = 320000
D_IN = 128
D_HID = 128
D_OUT = 2
EPS = 1e-5

NC = 2
NS = 16
CH = 128
NCHUNK = 81
EPT = CH * NCHUNK
RC = E // CH
WFULL = RC // NCHUNK
BEDGE = (RC - WFULL * NCHUNK) * CH
NPAD = N
ZROWS = 632
ZTAIL = NPAD - 15 * ZROWS
OROWS = 624
OTAIL = N - OROWS * NS

_f32 = jnp.float32
_mesh = plsc.VectorSubcoreMesh(core_axis_name="c", subcore_axis_name="s")


def _make_prop(width, nbuf, ring):
  ng = NCHUNK // nbuf
  assert ng * nbuf == NCHUNK
  idx_shape = (2 * nbuf * CH,) if ring else (EPT,)

  @functools.partial(
      pl.kernel,
      out_type=jax.ShapeDtypeStruct((NC, N, width), _f32),
      mesh=_mesh,
      compiler_params=pltpu.CompilerParams(use_tc_tiling_on_sc=False, skip_device_barrier=True),
      scratch_types=[
          pltpu.VMEM_SHARED((NPAD, width), _f32),
          pltpu.VMEM(idx_shape, jnp.int32),
          pltpu.VMEM(idx_shape, jnp.int32),
          [pltpu.VMEM((CH, width), _f32)] * nbuf,
          [pltpu.SemaphoreType.DMA] * nbuf,
          [pltpu.SemaphoreType.DMA] * nbuf,
          [pltpu.SemaphoreType.DMA] * 2,
      ],
  )
  def prop(src_hbm, dst_hbm, rows_hbm, zeros_hbm, out_hbm,
           acc, srcb, dstb, rows, gsem, ssem, isem):
    c = lax.axis_index("c")
    s = lax.axis_index("s")
    wid = c * NS + s
    grp0 = wid * NCHUNK
    e0 = wid * EPT

    def src_row(j, r):
      if ring:
        return srcb.at[pl.ds((lax.rem(j, 2) * nbuf + r) * CH, CH)]
      return srcb.at[pl.ds((j * nbuf + r) * CH, CH)]

    def dst_row(j, r):
      if ring:
        return dstb.at[pl.ds((lax.rem(j, 2) * nbuf + r) * CH, CH)]
      return dstb.at[pl.ds((j * nbuf + r) * CH, CH)]

    if ring:
      for r in range(nbuf):

        @pl.when(grp0 + r < RC)
        def _():
          pltpu.sync_copy(src_hbm.at[pl.ds(e0 + r * CH, CH)],
                          srcb.at[pl.ds(r * CH, CH)])
          pltpu.sync_copy(dst_hbm.at[pl.ds(e0 + r * CH, CH)],
                          dstb.at[pl.ds(r * CH, CH)])
    else:
      @pl.when(wid < WFULL)
      def _():
        pltpu.sync_copy(src_hbm.at[pl.ds(e0, EPT)], srcb)
        pltpu.sync_copy(dst_hbm.at[pl.ds(e0, EPT)], dstb)

      @pl.when(wid == WFULL)
      def _():
        pltpu.sync_copy(src_hbm.at[pl.ds(WFULL * EPT, BEDGE)],
                        srcb.at[pl.ds(0, BEDGE)])
        pltpu.sync_copy(dst_hbm.at[pl.ds(WFULL * EPT, BEDGE)],
                        dstb.at[pl.ds(0, BEDGE)])

    @pl.when(s < NS - 1)
    def _():
      pltpu.sync_copy(zeros_hbm, acc.at[pl.ds(s * ZROWS, ZROWS)])

    @pl.when(s == NS - 1)
    def _():
      pltpu.sync_copy(zeros_hbm.at[pl.ds(0, ZTAIL)],
                      acc.at[pl.ds((NS - 1) * ZROWS, ZTAIL)])

    plsc.subcore_barrier()

    for r in range(nbuf):

      @pl.when(grp0 + r < RC)
      def _():
        pltpu.async_copy(rows_hbm.at[src_row(0, r)], rows[r], gsem[r])

    def group(j, carry):
      q = 1 - lax.rem(j, 2)
      if ring:
        for r in range(nbuf):
          kn = grp0 + (j + 1) * nbuf + r

          @pl.when(jnp.logical_and(j < ng - 1, kn < RC))
          def _():
            en = e0 + ((j + 1) * nbuf + r) * CH
            off = (q * nbuf + r) * CH
            pltpu.async_copy(src_hbm.at[pl.ds(en, CH)],
                             srcb.at[pl.ds(off, CH)], isem[0])
            pltpu.async_copy(dst_hbm.at[pl.ds(en, CH)],
                             dstb.at[pl.ds(off, CH)], isem[1])

      for r in range(nbuf):
        kg = grp0 + j * nbuf + r

        @pl.when(kg < RC)
        def _():
          pltpu.make_async_copy(rows_hbm.at[src_row(j, r)],
                                rows[r], gsem[r]).wait()
          pltpu.async_copy(rows[r], acc.at[dst_row(j, r)], ssem[r], add=True)

      if ring:
        for r in range(nbuf):
          kn = grp0 + (j + 1) * nbuf + r

          @pl.when(jnp.logical_and(j < ng - 1, kn < RC))
          def _():
            en = e0 + ((j + 1) * nbuf + r) * CH
            off = (q * nbuf + r) * CH
            pltpu.make_async_copy(src_hbm.at[pl.ds(en, CH)],
                                  srcb.at[pl.ds(off, CH)], isem[0]).wait()
            pltpu.make_async_copy(dst_hbm.at[pl.ds(en, CH)],
                                  dstb.at[pl.ds(off, CH)], isem[1]).wait()

      for r in range(nbuf):
        kg = grp0 + j * nbuf + r

        @pl.when(kg < RC)
        def _():
          pltpu.make_async_copy(rows[r], acc.at[dst_row(j, r)],
                                ssem[r]).wait()

        @pl.when(jnp.logical_and(j < ng - 1, kg + nbuf < RC))
        def _():
          pltpu.async_copy(rows_hbm.at[src_row(j + 1, r)], rows[r], gsem[r])
      return carry

    lax.fori_loop(0, ng, group, 0)
    plsc.subcore_barrier()
    pltpu.sync_copy(acc.at[pl.ds(s * OROWS, OROWS)],
                    out_hbm.at[c, pl.ds(s * OROWS, OROWS)])

    @pl.when(s == NS - 1)
    def _():
      pltpu.sync_copy(acc.at[pl.ds(NS * OROWS, OTAIL)],
                      out_hbm.at[c, pl.ds(NS * OROWS, OTAIL)])

  return prop


@functools.partial(
    pl.kernel,
    out_type=jax.ShapeDtypeStruct((NC, N, 8), _f32),
    mesh=_mesh,
    compiler_params=pltpu.CompilerParams(use_tc_tiling_on_sc=False, skip_device_barrier=True),
    scratch_types=[
        pltpu.VMEM_SHARED((NPAD, 8), _f32),
        pltpu.VMEM((EPT,), jnp.int32),
        pltpu.VMEM((CH, 8), _f32),
        [pltpu.SemaphoreType.DMA] * 9,
    ],
)
def _deg_kernel(dst_hbm, ones_hbm, zeros_hbm, out_hbm, acc, dstb, ones_v,
                ssem):
  nbuf = 9
  ng = NCHUNK // nbuf
  c = lax.axis_index("c")
  s = lax.axis_index("s")
  wid = c * NS + s

  @pl.when(wid < WFULL)
  def _():
    pltpu.sync_copy(dst_hbm.at[pl.ds(wid * EPT, EPT)], dstb)

  @pl.when(wid == WFULL)
  def _():
    pltpu.sync_copy(dst_hbm.at[pl.ds(WFULL * EPT, BEDGE)],
                    dstb.at[pl.ds(0, BEDGE)])

  @pl.when(s < NS - 1)
  def _():
    pltpu.sync_copy(zeros_hbm, acc.at[pl.ds(s * ZROWS, ZROWS)])

  @pl.when(s == NS - 1)
  def _():
    pltpu.sync_copy(zeros_hbm.at[pl.ds(0, ZTAIL)],
                    acc.at[pl.ds((NS - 1) * ZROWS, ZTAIL)])

  pltpu.sync_copy(ones_hbm, ones_v)
  plsc.subcore_barrier()

  grp0 = wid * NCHUNK

  def _dsl(k):
    return dstb.at[pl.ds(k * CH, CH)]

  def group(j, carry):
    k0 = j * nbuf
    for r in range(nbuf):

      @pl.when(jnp.logical_and(j > 0, grp0 + k0 - nbuf + r < RC))
      def _():
        pltpu.make_async_copy(ones_v, acc.at[_dsl(k0 - nbuf + r)],
                              ssem[r]).wait()

      @pl.when(grp0 + k0 + r < RC)
      def _():
        pltpu.async_copy(ones_v, acc.at[_dsl(k0 + r)], ssem[r], add=True)

    return carry

  lax.fori_loop(0, ng, group, 0)
  for r in range(nbuf):

    @pl.when(grp0 + (ng - 1) * nbuf + r < RC)
    def _():
      pltpu.make_async_copy(ones_v, acc.at[_dsl((ng - 1) * nbuf + r)],
                            ssem[r]).wait()

  plsc.subcore_barrier()
  pltpu.sync_copy(acc.at[pl.ds(s * OROWS, OROWS)],
                  out_hbm.at[c, pl.ds(s * OROWS, OROWS)])

  @pl.when(s == NS - 1)
  def _():
    pltpu.sync_copy(acc.at[pl.ds(NS * OROWS, OTAIL)],
                    out_hbm.at[c, pl.ds(NS * OROWS, OTAIL)])


_prop128 = _make_prop(D_HID, 3, True)
_prop8 = _make_prop(8, 9, False)

BLK = 1000
GRID = N // BLK


def _dinv_of(deg_ref):
  deg = deg_ref[0, :, 0] + deg_ref[1, :, 0] + 1.0
  return lax.rsqrt(deg)


def _prep1_body(deg_ref, u_ref, w1_ref, g1_ref):
  dinv = _dinv_of(deg_ref)
  h = jnp.dot(u_ref[...], w1_ref[...], preferred_element_type=_f32)
  g1_ref[...] = h * dinv[:, None]


def _s_block(deg_ref, acc_ref, g_ref, b_ref):
  dinv = _dinv_of(deg_ref)
  s = dinv[:, None] * (acc_ref[0] + acc_ref[1] + g_ref[...]) + b_ref[...]
  return dinv, s


def _accum_stats(st_ref, s):
  pid = pl.program_id(0)

  @pl.when(pid == 0)
  def _():
    st_ref[...] = jnp.zeros_like(st_ref)

  st_ref[...] += jnp.concatenate(
      [jnp.sum(s, axis=0)[None], jnp.sum(s * s, axis=0)[None],
       jnp.zeros((6, s.shape[1]), _f32)], axis=0)


def _normed(st_ref, s):
  mean = st_ref[0:1, :] / N
  var = st_ref[1:2, :] / N - mean * mean
  return (s - mean) * lax.rsqrt(var + EPS)


def _bn1_body(deg_ref, acc_ref, g1_ref, b1_ref, gam_ref, bet_ref, w2_ref,
              g2_ref, st_ref):
  pid = pl.program_id(0)
  dinv, s1 = _s_block(deg_ref, acc_ref, g1_ref, b1_ref)

  @pl.when(pid < GRID)
  def _():
    _accum_stats(st_ref, s1)

  @pl.when(pid >= GRID)
  def _():
    r = jnp.maximum(gam_ref[...] * _normed(st_ref, s1) + bet_ref[...], 0.0)
    h2 = jnp.dot(r, w2_ref[...], preferred_element_type=_f32)
    g2_ref[...] = h2 * dinv[:, None]


def _bn2_body(deg_ref, acc_ref, g2_ref, b2_ref, gam_ref, bet_ref,
              sig_ref, sm_ref, st_ref):
  pid = pl.program_id(0)
  _, s2 = _s_block(deg_ref, acc_ref, g2_ref, b2_ref)

  @pl.when(pid < GRID)
  def _():
    _accum_stats(st_ref, s2)

  @pl.when(pid >= GRID)
  def _():
    y = gam_ref[...] * _normed(st_ref, s2) + bet_ref[...]
    sig_ref[...] = (1.0 / (1.0 + jnp.exp(-y)))[:, :D_OUT]
    a = y[:, 0:1]
    b = y[:, 1:2]
    m = jnp.maximum(a, b)
    ea = jnp.exp(a - m)
    eb = jnp.exp(b - m)
    tot = ea + eb
    sm_ref[...] = jnp.concatenate([ea / tot, eb / tot], axis=1)


def _row_spec(width):
  return pl.BlockSpec((BLK, width), lambda i: (i, 0))


_deg_spec = pl.BlockSpec((NC, BLK, 8), lambda i: (0, i, 0))
_full = lambda shape: pl.BlockSpec(shape, lambda i: tuple(0 for _ in shape))

_prep1 = pl.pallas_call(
    _prep1_body,
    grid=(GRID,),
    in_specs=[_deg_spec, _row_spec(D_IN), _full((D_IN, D_HID))],
    out_specs=_row_spec(D_HID),
    out_shape=jax.ShapeDtypeStruct((N, D_HID), _f32),
)

_row_spec2 = lambda width: pl.BlockSpec((BLK, width), lambda i: (i % GRID, 0))
_deg_spec2 = pl.BlockSpec((NC, BLK, 8), lambda i: (0, i % GRID, 0))

_bn1 = pl.pallas_call(
    _bn1_body,
    grid=(2 * GRID,),
    in_specs=[_deg_spec2,
              pl.BlockSpec((NC, BLK, D_HID), lambda i: (0, i % GRID, 0)),
              _row_spec2(D_HID), _full((1, D_HID)), _full((1, D_HID)),
              _full((1, D_HID)), _full((D_HID, 8))],
    out_specs=_row_spec2(8),
    out_shape=jax.ShapeDtypeStruct((N, 8), _f32),
    scratch_shapes=[pltpu.VMEM((8, D_HID), _f32)],
)

_bn2 = pl.pallas_call(
    _bn2_body,
    grid=(2 * GRID,),
    in_specs=[_deg_spec2,
              pl.BlockSpec((NC, BLK, 8), lambda i: (0, i % GRID, 0)),
              _row_spec2(8), _full((1, 8)), _full((1, 8)), _full((1, 8))],
    out_specs=[_row_spec2(D_OUT), _row_spec2(D_OUT)],
    out_shape=[jax.ShapeDtypeStruct((N, D_OUT), _f32),
               jax.ShapeDtypeStruct((N, D_OUT), _f32)],
    scratch_shapes=[pltpu.VMEM((8, 8), _f32)],
)


@jax.jit
def kernel(edge_index, u_S, W1, b1, gamma1, beta1, W2, b2, gamma2, beta2):
  src = edge_index[0]
  dst = edge_index[1]

  ones8 = jnp.ones((CH, 8), _f32)
  zeros8 = jnp.zeros((ZROWS, 8), _f32)
  zeros128 = jnp.zeros((ZROWS, D_HID), _f32)

  deg2 = _deg_kernel(dst, ones8, zeros8)

  w1 = W1.astype(_f32)
  g1 = _prep1(deg2, u_S, w1)
  acc1 = _prop128(src, dst, g1, zeros128)

  b1r = b1.reshape(1, D_HID)
  w2p = jnp.concatenate([W2, jnp.zeros((D_HID, 8 - D_OUT), _f32)], axis=1)
  g2 = _bn1(deg2, acc1, g1, b1r, gamma1.reshape(1, -1),
            beta1.reshape(1, -1), w2p)

  acc2 = _prop8(src, dst, g2, zeros8)

  pad2 = lambda v: jnp.concatenate([v, jnp.zeros((8 - D_OUT,), _f32)]).reshape(1, 8)
  sig, sm = _bn2(deg2, acc2, g2, pad2(b2), pad2(gamma2), pad2(beta2))
  return sig, sm


# speedup vs baseline: 1.1009x; 1.1009x over previous
import functools

import jax
import jax.numpy as jnp
from jax import lax
from jax.experimental import pallas as pl
from jax.experimental.pallas import tpu as pltpu
from jax.experimental.pallas import tpu_sc as plsc

N = 10000
E = 320000
D_IN = 128
D_HID = 128
D_OUT = 2
EPS = 1e-5

NC = 2
NS = 16
CH = 128
NCHUNK = 81
EPT = CH * NCHUNK
RC = E // CH
WFULL = RC // NCHUNK
BEDGE = (RC - WFULL * NCHUNK) * CH
NPAD = N
ZROWS = 632
ZTAIL = NPAD - 15 * ZROWS
OROWS = 624
OTAIL = N - OROWS * NS

_f32 = jnp.float32
_mesh = plsc.VectorSubcoreMesh(core_axis_name="c", subcore_axis_name="s")


def _make_prop(width, nbuf, ring):
  ng = NCHUNK // nbuf
  assert ng * nbuf == NCHUNK
  idx_shape = (2 * nbuf * CH,) if ring else (EPT,)

  @functools.partial(
      pl.kernel,
      out_type=jax.ShapeDtypeStruct((NC, N, width), _f32),
      mesh=_mesh,
      compiler_params=pltpu.CompilerParams(use_tc_tiling_on_sc=False, skip_device_barrier=True),
      scratch_types=[
          pltpu.VMEM_SHARED((NPAD, width), _f32),
          pltpu.VMEM(idx_shape, jnp.int32),
          pltpu.VMEM(idx_shape, jnp.int32),
          [pltpu.VMEM((CH, width), _f32)] * nbuf,
          [pltpu.SemaphoreType.DMA] * nbuf,
          [pltpu.SemaphoreType.DMA] * nbuf,
          [pltpu.SemaphoreType.DMA] * 2,
      ],
  )
  def prop(ei_hbm, rows_hbm, zeros_hbm, out_hbm,
           acc, srcb, dstb, rows, gsem, ssem, isem):
    src_hbm = ei_hbm.at[0]
    dst_hbm = ei_hbm.at[1]
    c = lax.axis_index("c")
    s = lax.axis_index("s")
    wid = c * NS + s
    grp0 = wid * NCHUNK
    e0 = wid * EPT

    def src_row(j, r):
      if ring:
        return srcb.at[pl.ds((lax.rem(j, 2) * nbuf + r) * CH, CH)]
      return srcb.at[pl.ds((j * nbuf + r) * CH, CH)]

    def dst_row(j, r):
      if ring:
        return dstb.at[pl.ds((lax.rem(j, 2) * nbuf + r) * CH, CH)]
      return dstb.at[pl.ds((j * nbuf + r) * CH, CH)]

    if ring:
      for r in range(nbuf):

        @pl.when(grp0 + r < RC)
        def _():
          pltpu.sync_copy(src_hbm.at[pl.ds(e0 + r * CH, CH)],
                          srcb.at[pl.ds(r * CH, CH)])
          pltpu.sync_copy(dst_hbm.at[pl.ds(e0 + r * CH, CH)],
                          dstb.at[pl.ds(r * CH, CH)])
    else:
      @pl.when(wid < WFULL)
      def _():
        pltpu.sync_copy(src_hbm.at[pl.ds(e0, EPT)], srcb)
        pltpu.sync_copy(dst_hbm.at[pl.ds(e0, EPT)], dstb)

      @pl.when(wid == WFULL)
      def _():
        pltpu.sync_copy(src_hbm.at[pl.ds(WFULL * EPT, BEDGE)],
                        srcb.at[pl.ds(0, BEDGE)])
        pltpu.sync_copy(dst_hbm.at[pl.ds(WFULL * EPT, BEDGE)],
                        dstb.at[pl.ds(0, BEDGE)])

    @pl.when(s < NS - 1)
    def _():
      pltpu.sync_copy(zeros_hbm, acc.at[pl.ds(s * ZROWS, ZROWS)])

    @pl.when(s == NS - 1)
    def _():
      pltpu.sync_copy(zeros_hbm.at[pl.ds(0, ZTAIL)],
                      acc.at[pl.ds((NS - 1) * ZROWS, ZTAIL)])

    plsc.subcore_barrier()

    for r in range(nbuf):

      @pl.when(grp0 + r < RC)
      def _():
        pltpu.async_copy(rows_hbm.at[src_row(0, r)], rows[r], gsem[r])

    def group(j, carry):
      q = 1 - lax.rem(j, 2)
      if ring:
        for r in range(nbuf):
          kn = grp0 + (j + 1) * nbuf + r

          @pl.when(jnp.logical_and(j < ng - 1, kn < RC))
          def _():
            en = e0 + ((j + 1) * nbuf + r) * CH
            off = (q * nbuf + r) * CH
            pltpu.async_copy(src_hbm.at[pl.ds(en, CH)],
                             srcb.at[pl.ds(off, CH)], isem[0])
            pltpu.async_copy(dst_hbm.at[pl.ds(en, CH)],
                             dstb.at[pl.ds(off, CH)], isem[1])

      for r in range(nbuf):
        kg = grp0 + j * nbuf + r

        @pl.when(kg < RC)
        def _():
          pltpu.make_async_copy(rows_hbm.at[src_row(j, r)],
                                rows[r], gsem[r]).wait()
          pltpu.async_copy(rows[r], acc.at[dst_row(j, r)], ssem[r], add=True)

      if ring:
        for r in range(nbuf):
          kn = grp0 + (j + 1) * nbuf + r

          @pl.when(jnp.logical_and(j < ng - 1, kn < RC))
          def _():
            en = e0 + ((j + 1) * nbuf + r) * CH
            off = (q * nbuf + r) * CH
            pltpu.make_async_copy(src_hbm.at[pl.ds(en, CH)],
                                  srcb.at[pl.ds(off, CH)], isem[0]).wait()
            pltpu.make_async_copy(dst_hbm.at[pl.ds(en, CH)],
                                  dstb.at[pl.ds(off, CH)], isem[1]).wait()

      for r in range(nbuf):
        kg = grp0 + j * nbuf + r

        @pl.when(kg < RC)
        def _():
          pltpu.make_async_copy(rows[r], acc.at[dst_row(j, r)],
                                ssem[r]).wait()

        @pl.when(jnp.logical_and(j < ng - 1, kg + nbuf < RC))
        def _():
          pltpu.async_copy(rows_hbm.at[src_row(j + 1, r)], rows[r], gsem[r])
      return carry

    lax.fori_loop(0, ng, group, 0)
    plsc.subcore_barrier()
    pltpu.sync_copy(acc.at[pl.ds(s * OROWS, OROWS)],
                    out_hbm.at[c, pl.ds(s * OROWS, OROWS)])

    @pl.when(s == NS - 1)
    def _():
      pltpu.sync_copy(acc.at[pl.ds(NS * OROWS, OTAIL)],
                      out_hbm.at[c, pl.ds(NS * OROWS, OTAIL)])

  return prop


@functools.partial(
    pl.kernel,
    out_type=jax.ShapeDtypeStruct((NC, N, 8), _f32),
    mesh=_mesh,
    compiler_params=pltpu.CompilerParams(use_tc_tiling_on_sc=False, skip_device_barrier=True),
    scratch_types=[
        pltpu.VMEM_SHARED((NPAD, 8), _f32),
        pltpu.VMEM((EPT,), jnp.int32),
        pltpu.VMEM((CH, 8), _f32),
        [pltpu.SemaphoreType.DMA] * 9,
    ],
)
def _deg_kernel(ei_hbm, ones_hbm, zeros_hbm, out_hbm, acc, dstb, ones_v,
                ssem):
  dst_hbm = ei_hbm.at[1]
  nbuf = 9
  ng = NCHUNK // nbuf
  c = lax.axis_index("c")
  s = lax.axis_index("s")
  wid = c * NS + s

  @pl.when(wid < WFULL)
  def _():
    pltpu.sync_copy(dst_hbm.at[pl.ds(wid * EPT, EPT)], dstb)

  @pl.when(wid == WFULL)
  def _():
    pltpu.sync_copy(dst_hbm.at[pl.ds(WFULL * EPT, BEDGE)],
                    dstb.at[pl.ds(0, BEDGE)])

  @pl.when(s < NS - 1)
  def _():
    pltpu.sync_copy(zeros_hbm, acc.at[pl.ds(s * ZROWS, ZROWS)])

  @pl.when(s == NS - 1)
  def _():
    pltpu.sync_copy(zeros_hbm.at[pl.ds(0, ZTAIL)],
                    acc.at[pl.ds((NS - 1) * ZROWS, ZTAIL)])

  pltpu.sync_copy(ones_hbm, ones_v)
  plsc.subcore_barrier()

  grp0 = wid * NCHUNK

  def _dsl(k):
    return dstb.at[pl.ds(k * CH, CH)]

  def group(j, carry):
    k0 = j * nbuf
    for r in range(nbuf):

      @pl.when(jnp.logical_and(j > 0, grp0 + k0 - nbuf + r < RC))
      def _():
        pltpu.make_async_copy(ones_v, acc.at[_dsl(k0 - nbuf + r)],
                              ssem[r]).wait()

      @pl.when(grp0 + k0 + r < RC)
      def _():
        pltpu.async_copy(ones_v, acc.at[_dsl(k0 + r)], ssem[r], add=True)

    return carry

  lax.fori_loop(0, ng, group, 0)
  for r in range(nbuf):

    @pl.when(grp0 + (ng - 1) * nbuf + r < RC)
    def _():
      pltpu.make_async_copy(ones_v, acc.at[_dsl((ng - 1) * nbuf + r)],
                            ssem[r]).wait()

  plsc.subcore_barrier()
  pltpu.sync_copy(acc.at[pl.ds(s * OROWS, OROWS)],
                  out_hbm.at[c, pl.ds(s * OROWS, OROWS)])

  @pl.when(s == NS - 1)
  def _():
    pltpu.sync_copy(acc.at[pl.ds(NS * OROWS, OTAIL)],
                    out_hbm.at[c, pl.ds(NS * OROWS, OTAIL)])


_prop128 = _make_prop(D_HID, 3, True)
_prop8 = _make_prop(8, 9, False)

BLK = 2000
GRID = N // BLK
G1 = GRID


def _dinv_of(deg_ref):
  deg = deg_ref[0, :, 0] + deg_ref[1, :, 0] + 1.0
  return lax.rsqrt(deg)


def _prep1_body(deg_ref, u_ref, w1_ref, g1_ref):
  dinv = _dinv_of(deg_ref)
  h = jnp.dot(u_ref[...], w1_ref[...], preferred_element_type=_f32)
  g1_ref[...] = h * dinv[:, None]


def _s_block(deg_ref, acc_ref, g_ref, b_ref):
  dinv = _dinv_of(deg_ref)
  s = dinv[:, None] * (acc_ref[0] + acc_ref[1] + g_ref[...]) + b_ref[...]
  return dinv, s


def _accum_stats(st_ref, s):
  pid = pl.program_id(0)

  @pl.when(pid == 0)
  def _():
    st_ref[...] = jnp.zeros_like(st_ref)

  st_ref[...] += jnp.concatenate(
      [jnp.sum(s, axis=0)[None], jnp.sum(s * s, axis=0)[None],
       jnp.zeros((6, s.shape[1]), _f32)], axis=0)


def _normed(st_ref, s):
  mean = st_ref[0:1, :] / N
  var = st_ref[1:2, :] / N - mean * mean
  return (s - mean) * lax.rsqrt(var + EPS)


def _bn1_body(deg_ref, acc_ref, g1_ref, b1_ref, gam_ref, bet_ref, w2_ref,
              g2_ref, st_ref):
  pid = pl.program_id(0)
  dinv, s1 = _s_block(deg_ref, acc_ref, g1_ref, b1_ref)

  @pl.when(pid < G1)
  def _():
    _accum_stats(st_ref, s1)

  @pl.when(pid >= G1)
  def _():
    r = jnp.maximum(gam_ref[...] * _normed(st_ref, s1) + bet_ref[...], 0.0)
    h2 = jnp.dot(r, w2_ref[...], preferred_element_type=_f32)
    g2_ref[...] = h2 * dinv[:, None]


def _bn2_body(deg_ref, acc_ref, g2_ref, b2_ref, gam_ref, bet_ref,
              sig_ref, sm_ref, st_ref):
  pid = pl.program_id(0)
  _, s2 = _s_block(deg_ref, acc_ref, g2_ref, b2_ref)

  @pl.when(pid == 0)
  def _():
    _accum_stats(st_ref, s2)

  @pl.when(pid == 1)
  def _():
    y = gam_ref[...] * _normed(st_ref, s2) + bet_ref[...]
    sig_ref[...] = (1.0 / (1.0 + jnp.exp(-y)))[:, :D_OUT]
    a = y[:, 0:1]
    b = y[:, 1:2]
    m = jnp.maximum(a, b)
    ea = jnp.exp(a - m)
    eb = jnp.exp(b - m)
    tot = ea + eb
    sm_ref[...] = jnp.concatenate([ea / tot, eb / tot], axis=1)


def _row_spec(width):
  return pl.BlockSpec((BLK, width), lambda i: (i, 0))


_deg_spec = pl.BlockSpec((NC, BLK, 8), lambda i: (0, i, 0))
_full = lambda shape: pl.BlockSpec(shape, lambda i: tuple(0 for _ in shape))

_prep1 = pl.pallas_call(
    _prep1_body,
    grid=(GRID,),
    in_specs=[_deg_spec, _row_spec(D_IN), _full((D_IN, D_HID))],
    out_specs=_row_spec(D_HID),
    out_shape=jax.ShapeDtypeStruct((N, D_HID), _f32),
)

_row_spec2 = lambda width: pl.BlockSpec((BLK, width), lambda i: (i % G1, 0))
_deg_spec2 = pl.BlockSpec((NC, BLK, 8), lambda i: (0, i % G1, 0))

_bn1 = pl.pallas_call(
    _bn1_body,
    grid=(2 * G1,),
    in_specs=[_deg_spec2,
              pl.BlockSpec((NC, BLK, D_HID), lambda i: (0, i % G1, 0)),
              _row_spec2(D_HID), _full((1, D_HID)), _full((1, D_HID)),
              _full((1, D_HID)), _full((D_HID, 8))],
    out_specs=_row_spec2(8),
    out_shape=jax.ShapeDtypeStruct((N, 8), _f32),
    scratch_shapes=[pltpu.VMEM((8, D_HID), _f32)],
)

_nfull = lambda width: pl.BlockSpec((N, width), lambda i: (0, 0))

_bn2 = pl.pallas_call(
    _bn2_body,
    grid=(2,),
    in_specs=[pl.BlockSpec((NC, N, 8), lambda i: (0, 0, 0)),
              pl.BlockSpec((NC, N, 8), lambda i: (0, 0, 0)),
              _nfull(8), _full((1, 8)), _full((1, 8)), _full((1, 8))],
    out_specs=[_nfull(D_OUT), _nfull(D_OUT)],
    out_shape=[jax.ShapeDtypeStruct((N, D_OUT), _f32),
               jax.ShapeDtypeStruct((N, D_OUT), _f32)],
    scratch_shapes=[pltpu.VMEM((8, 8), _f32)],
)


@jax.jit
def kernel(edge_index, u_S, W1, b1, gamma1, beta1, W2, b2, gamma2, beta2):
  ones8 = jnp.ones((CH, 8), _f32)
  zeros8 = jnp.zeros((ZROWS, 8), _f32)
  zeros128 = jnp.zeros((ZROWS, D_HID), _f32)

  deg2 = _deg_kernel(edge_index, ones8, zeros8)

  w1 = W1.astype(_f32)
  g1 = _prep1(deg2, u_S, w1)
  acc1 = _prop128(edge_index, g1, zeros128)

  b1r = b1.reshape(1, D_HID)
  w2p = jnp.concatenate([W2, jnp.zeros((D_HID, 8 - D_OUT), _f32)], axis=1)
  g2 = _bn1(deg2, acc1, g1, b1r, gamma1.reshape(1, -1),
            beta1.reshape(1, -1), w2p)

  acc2 = _prop8(edge_index, g2, zeros8)

  pad2 = lambda v: jnp.concatenate([v, jnp.zeros((8 - D_OUT,), _f32)]).reshape(1, 8)
  sig, sm = _bn2(deg2, acc2, g2, pad2(b2), pad2(gamma2), pad2(beta2))
  return sig, sm


# speedup vs baseline: 1.1262x; 1.0230x over previous
import functools

import jax
import jax.numpy as jnp
from jax import lax
from jax.experimental import pallas as pl
from jax.experimental.pallas import tpu as pltpu
from jax.experimental.pallas import tpu_sc as plsc

N = 10000
E = 320000
D_IN = 128
D_HID = 128
D_OUT = 2
EPS = 1e-5

NC = 2
NS = 16
CH = 128
NCHUNK = 81
EPT = CH * NCHUNK
RC = E // CH
WFULL = RC // NCHUNK
BEDGE = (RC - WFULL * NCHUNK) * CH
NPAD = N
ZROWS = 632
ZTAIL = NPAD - 15 * ZROWS
OROWS = 624
OTAIL = N - OROWS * NS

_f32 = jnp.float32
_mesh = plsc.VectorSubcoreMesh(core_axis_name="c", subcore_axis_name="s")


def _make_prop(width, nbuf, ring):
  ng = NCHUNK // nbuf
  assert ng * nbuf == NCHUNK
  idx_shape = (2 * nbuf * CH,) if ring else (EPT,)

  @functools.partial(
      pl.kernel,
      out_type=jax.ShapeDtypeStruct((NC, N, width), _f32),
      mesh=_mesh,
      compiler_params=pltpu.CompilerParams(use_tc_tiling_on_sc=False, skip_device_barrier=True),
      scratch_types=[
          pltpu.VMEM_SHARED((NPAD, width), _f32),
          pltpu.VMEM(idx_shape, jnp.int32),
          pltpu.VMEM(idx_shape, jnp.int32),
          [pltpu.VMEM((CH, width), _f32)] * nbuf,
          [pltpu.SemaphoreType.DMA] * nbuf,
          [pltpu.SemaphoreType.DMA] * nbuf,
          [pltpu.SemaphoreType.DMA] * 2,
      ],
  )
  def prop(ei_hbm, rows_hbm, zeros_hbm, out_hbm,
           acc, srcb, dstb, rows, gsem, ssem, isem):
    src_hbm = ei_hbm.at[0]
    dst_hbm = ei_hbm.at[1]
    c = lax.axis_index("c")
    s = lax.axis_index("s")
    wid = c * NS + s
    grp0 = wid * NCHUNK
    e0 = wid * EPT

    def src_row(j, r):
      if ring:
        return srcb.at[pl.ds((lax.rem(j, 2) * nbuf + r) * CH, CH)]
      return srcb.at[pl.ds((j * nbuf + r) * CH, CH)]

    def dst_row(j, r):
      if ring:
        return dstb.at[pl.ds((lax.rem(j, 2) * nbuf + r) * CH, CH)]
      return dstb.at[pl.ds((j * nbuf + r) * CH, CH)]

    if ring:
      for r in range(nbuf):

        @pl.when(grp0 + r < RC)
        def _():
          pltpu.sync_copy(src_hbm.at[pl.ds(e0 + r * CH, CH)],
                          srcb.at[pl.ds(r * CH, CH)])
          pltpu.sync_copy(dst_hbm.at[pl.ds(e0 + r * CH, CH)],
                          dstb.at[pl.ds(r * CH, CH)])
    else:
      @pl.when(wid < WFULL)
      def _():
        pltpu.sync_copy(src_hbm.at[pl.ds(e0, EPT)], srcb)
        pltpu.sync_copy(dst_hbm.at[pl.ds(e0, EPT)], dstb)

      @pl.when(wid == WFULL)
      def _():
        pltpu.sync_copy(src_hbm.at[pl.ds(WFULL * EPT, BEDGE)],
                        srcb.at[pl.ds(0, BEDGE)])
        pltpu.sync_copy(dst_hbm.at[pl.ds(WFULL * EPT, BEDGE)],
                        dstb.at[pl.ds(0, BEDGE)])

    @pl.when(s < NS - 1)
    def _():
      pltpu.sync_copy(zeros_hbm, acc.at[pl.ds(s * ZROWS, ZROWS)])

    @pl.when(s == NS - 1)
    def _():
      pltpu.sync_copy(zeros_hbm.at[pl.ds(0, ZTAIL)],
                      acc.at[pl.ds((NS - 1) * ZROWS, ZTAIL)])

    plsc.subcore_barrier()

    for r in range(nbuf):

      @pl.when(grp0 + r < RC)
      def _():
        pltpu.async_copy(rows_hbm.at[src_row(0, r)], rows[r], gsem[r])

    def group(j, carry):
      q = 1 - lax.rem(j, 2)
      if ring:
        for r in range(nbuf):
          kn = grp0 + (j + 1) * nbuf + r

          @pl.when(jnp.logical_and(j < ng - 1, kn < RC))
          def _():
            en = e0 + ((j + 1) * nbuf + r) * CH
            off = (q * nbuf + r) * CH
            pltpu.async_copy(src_hbm.at[pl.ds(en, CH)],
                             srcb.at[pl.ds(off, CH)], isem[0])
            pltpu.async_copy(dst_hbm.at[pl.ds(en, CH)],
                             dstb.at[pl.ds(off, CH)], isem[1])

      for r in range(nbuf):
        kg = grp0 + j * nbuf + r

        @pl.when(kg < RC)
        def _():
          pltpu.make_async_copy(rows_hbm.at[src_row(j, r)],
                                rows[r], gsem[r]).wait()
          pltpu.async_copy(rows[r], acc.at[dst_row(j, r)], ssem[r], add=True)

      if ring:
        for r in range(nbuf):
          kn = grp0 + (j + 1) * nbuf + r

          @pl.when(jnp.logical_and(j < ng - 1, kn < RC))
          def _():
            en = e0 + ((j + 1) * nbuf + r) * CH
            off = (q * nbuf + r) * CH
            pltpu.make_async_copy(src_hbm.at[pl.ds(en, CH)],
                                  srcb.at[pl.ds(off, CH)], isem[0]).wait()
            pltpu.make_async_copy(dst_hbm.at[pl.ds(en, CH)],
                                  dstb.at[pl.ds(off, CH)], isem[1]).wait()

      for r in range(nbuf):
        kg = grp0 + j * nbuf + r

        @pl.when(kg < RC)
        def _():
          pltpu.make_async_copy(rows[r], acc.at[dst_row(j, r)],
                                ssem[r]).wait()

        @pl.when(jnp.logical_and(j < ng - 1, kg + nbuf < RC))
        def _():
          pltpu.async_copy(rows_hbm.at[src_row(j + 1, r)], rows[r], gsem[r])
      return carry

    lax.fori_loop(0, ng, group, 0)
    plsc.subcore_barrier()
    pltpu.sync_copy(acc.at[pl.ds(s * OROWS, OROWS)],
                    out_hbm.at[c, pl.ds(s * OROWS, OROWS)])

    @pl.when(s == NS - 1)
    def _():
      pltpu.sync_copy(acc.at[pl.ds(NS * OROWS, OTAIL)],
                      out_hbm.at[c, pl.ds(NS * OROWS, OTAIL)])

  return prop


@functools.partial(
    pl.kernel,
    out_type=jax.ShapeDtypeStruct((NC, N, 8), _f32),
    mesh=_mesh,
    compiler_params=pltpu.CompilerParams(use_tc_tiling_on_sc=False, skip_device_barrier=True),
    scratch_types=[
        pltpu.VMEM_SHARED((NPAD, 8), _f32),
        pltpu.VMEM((EPT,), jnp.int32),
        pltpu.VMEM((CH, 8), _f32),
        [pltpu.SemaphoreType.DMA] * 9,
    ],
)
def _deg_kernel(ei_hbm, ones_hbm, zeros_hbm, out_hbm, acc, dstb, ones_v,
                ssem):
  dst_hbm = ei_hbm.at[1]
  nbuf = 9
  ng = NCHUNK // nbuf
  c = lax.axis_index("c")
  s = lax.axis_index("s")
  wid = c * NS + s

  @pl.when(wid < WFULL)
  def _():
    pltpu.sync_copy(dst_hbm.at[pl.ds(wid * EPT, EPT)], dstb)

  @pl.when(wid == WFULL)
  def _():
    pltpu.sync_copy(dst_hbm.at[pl.ds(WFULL * EPT, BEDGE)],
                    dstb.at[pl.ds(0, BEDGE)])

  @pl.when(s < NS - 1)
  def _():
    pltpu.sync_copy(zeros_hbm, acc.at[pl.ds(s * ZROWS, ZROWS)])

  @pl.when(s == NS - 1)
  def _():
    pltpu.sync_copy(zeros_hbm.at[pl.ds(0, ZTAIL)],
                    acc.at[pl.ds((NS - 1) * ZROWS, ZTAIL)])

  pltpu.sync_copy(ones_hbm, ones_v)
  plsc.subcore_barrier()

  grp0 = wid * NCHUNK

  def _dsl(k):
    return dstb.at[pl.ds(k * CH, CH)]

  def group(j, carry):
    k0 = j * nbuf
    for r in range(nbuf):

      @pl.when(jnp.logical_and(j > 0, grp0 + k0 - nbuf + r < RC))
      def _():
        pltpu.make_async_copy(ones_v, acc.at[_dsl(k0 - nbuf + r)],
                              ssem[r]).wait()

      @pl.when(grp0 + k0 + r < RC)
      def _():
        pltpu.async_copy(ones_v, acc.at[_dsl(k0 + r)], ssem[r], add=True)

    return carry

  lax.fori_loop(0, ng, group, 0)
  for r in range(nbuf):

    @pl.when(grp0 + (ng - 1) * nbuf + r < RC)
    def _():
      pltpu.make_async_copy(ones_v, acc.at[_dsl((ng - 1) * nbuf + r)],
                            ssem[r]).wait()

  plsc.subcore_barrier()
  pltpu.sync_copy(acc.at[pl.ds(s * OROWS, OROWS)],
                  out_hbm.at[c, pl.ds(s * OROWS, OROWS)])

  @pl.when(s == NS - 1)
  def _():
    pltpu.sync_copy(acc.at[pl.ds(NS * OROWS, OTAIL)],
                    out_hbm.at[c, pl.ds(NS * OROWS, OTAIL)])


_prop128 = _make_prop(D_HID, 3, True)
_prop8 = _make_prop(8, 9, False)

BLK = 2000
GRID = N // BLK
G1 = GRID


def _dinv_of(deg_ref):
  deg = deg_ref[0, :, 0] + deg_ref[1, :, 0] + 1.0
  return lax.rsqrt(deg)


def _prep1_body(deg_ref, u_ref, w1_ref, g1_ref):
  dinv = _dinv_of(deg_ref)
  h = jnp.dot(u_ref[...], w1_ref[...], preferred_element_type=_f32)
  g1_ref[...] = h * dinv[:, None]


def _s_block(deg_ref, acc_ref, g_ref, b_ref):
  dinv = _dinv_of(deg_ref)
  s = dinv[:, None] * (acc_ref[0] + acc_ref[1] + g_ref[...]) + b_ref[...]
  return dinv, s


def _accum_stats(st_ref, s):
  pid = pl.program_id(0)

  @pl.when(pid == 0)
  def _():
    st_ref[...] = jnp.zeros_like(st_ref)

  st_ref[...] += jnp.concatenate(
      [jnp.sum(s, axis=0)[None], jnp.sum(s * s, axis=0)[None],
       jnp.zeros((6, s.shape[1]), _f32)], axis=0)


def _normed(st_ref, s):
  mean = st_ref[0:1, :] / N
  var = st_ref[1:2, :] / N - mean * mean
  return (s - mean) * lax.rsqrt(var + EPS)


def _bn1_body(deg_ref, acc_ref, g1_ref, b1_ref, gam_ref, bet_ref, w2_ref,
              g2_ref, st_ref, s1_scr, dinv_scr):
  pid = pl.program_id(0)

  @pl.when(pid < G1)
  def _():
    dinv, s1 = _s_block(deg_ref, acc_ref, g1_ref, b1_ref)
    _accum_stats(st_ref, s1)
    s1_scr[pl.ds(pid * BLK, BLK), :] = s1
    dinv_scr[pl.ds(pid * 8, 1), :] = dinv[None]

  @pl.when(pid >= G1)
  def _():
    i = pid - G1
    s1 = s1_scr[pl.ds(i * BLK, BLK), :]
    dinv = dinv_scr[pl.ds(i * 8, 1), :][0]
    r = jnp.maximum(gam_ref[...] * _normed(st_ref, s1) + bet_ref[...], 0.0)
    h2 = jnp.dot(r, w2_ref[...], preferred_element_type=_f32)
    g2_ref[...] = h2 * dinv[:, None]


def _bn2_body(deg_ref, acc_ref, g2_ref, b2_ref, gam_ref, bet_ref,
              sig_ref, sm_ref, st_ref, s2_scr):
  pid = pl.program_id(0)

  @pl.when(pid == 0)
  def _():
    _, s2 = _s_block(deg_ref, acc_ref, g2_ref, b2_ref)
    _accum_stats(st_ref, s2)
    s2_scr[...] = s2

  @pl.when(pid == 1)
  def _():
    s2 = s2_scr[...]
    y = gam_ref[...] * _normed(st_ref, s2) + bet_ref[...]
    sig_ref[...] = (1.0 / (1.0 + jnp.exp(-y)))[:, :D_OUT]
    a = y[:, 0:1]
    b = y[:, 1:2]
    m = jnp.maximum(a, b)
    ea = jnp.exp(a - m)
    eb = jnp.exp(b - m)
    tot = ea + eb
    sm_ref[...] = jnp.concatenate([ea / tot, eb / tot], axis=1)


def _row_spec(width):
  return pl.BlockSpec((BLK, width), lambda i: (i, 0))


_deg_spec = pl.BlockSpec((NC, BLK, 8), lambda i: (0, i, 0))
_full = lambda shape: pl.BlockSpec(shape, lambda i: tuple(0 for _ in shape))

_prep1 = pl.pallas_call(
    _prep1_body,
    grid=(GRID,),
    in_specs=[_deg_spec, _row_spec(D_IN), _full((D_IN, D_HID))],
    out_specs=_row_spec(D_HID),
    out_shape=jax.ShapeDtypeStruct((N, D_HID), _f32),
)

_row_spec2 = lambda width: pl.BlockSpec((BLK, width), lambda i: (i % G1, 0))
_deg_spec2 = pl.BlockSpec((NC, BLK, 8), lambda i: (0, i % G1, 0))

_pin = lambda i: jnp.where(i < G1, i, G1 - 1)

_bn1 = pl.pallas_call(
    _bn1_body,
    grid=(2 * G1,),
    in_specs=[pl.BlockSpec((NC, BLK, 8), lambda i: (0, _pin(i), 0)),
              pl.BlockSpec((NC, BLK, D_HID), lambda i: (0, _pin(i), 0)),
              pl.BlockSpec((BLK, D_HID), lambda i: (_pin(i), 0)),
              _full((1, D_HID)), _full((1, D_HID)),
              _full((1, D_HID)), _full((D_HID, 8))],
    out_specs=_row_spec2(8),
    out_shape=jax.ShapeDtypeStruct((N, 8), _f32),
    scratch_shapes=[pltpu.VMEM((8, D_HID), _f32),
                    pltpu.VMEM((N, D_HID), _f32),
                    pltpu.VMEM((G1 * 8, BLK), _f32)],
)

_nfull = lambda width: pl.BlockSpec((N, width), lambda i: (0, 0))

_bn2 = pl.pallas_call(
    _bn2_body,
    grid=(2,),
    in_specs=[pl.BlockSpec((NC, N, 8), lambda i: (0, 0, 0)),
              pl.BlockSpec((NC, N, 8), lambda i: (0, 0, 0)),
              _nfull(8), _full((1, 8)), _full((1, 8)), _full((1, 8))],
    out_specs=[_nfull(D_OUT), _nfull(D_OUT)],
    out_shape=[jax.ShapeDtypeStruct((N, D_OUT), _f32),
               jax.ShapeDtypeStruct((N, D_OUT), _f32)],
    scratch_shapes=[pltpu.VMEM((8, 8), _f32), pltpu.VMEM((N, 8), _f32)],
)


@jax.jit
def kernel(edge_index, u_S, W1, b1, gamma1, beta1, W2, b2, gamma2, beta2):
  ones8 = jnp.ones((CH, 8), _f32)
  zeros8 = jnp.zeros((ZROWS, 8), _f32)
  zeros128 = jnp.zeros((ZROWS, D_HID), _f32)

  deg2 = _deg_kernel(edge_index, ones8, zeros8)

  w1 = W1.astype(_f32)
  g1 = _prep1(deg2, u_S, w1)
  acc1 = _prop128(edge_index, g1, zeros128)

  b1r = b1.reshape(1, D_HID)
  w2p = jnp.concatenate([W2, jnp.zeros((D_HID, 8 - D_OUT), _f32)], axis=1)
  g2 = _bn1(deg2, acc1, g1, b1r, gamma1.reshape(1, -1),
            beta1.reshape(1, -1), w2p)

  acc2 = _prop8(edge_index, g2, zeros8)

  pad2 = lambda v: jnp.concatenate([v, jnp.zeros((8 - D_OUT,), _f32)]).reshape(1, 8)
  sig, sm = _bn2(deg2, acc2, g2, pad2(b2), pad2(gamma2), pad2(beta2))
  return sig, sm
